# baseline (speedup 1.0000x reference)
import functools

import jax
import jax.numpy as jnp
from jax import lax
from jax.experimental import pallas as pl
from jax.experimental.pallas import tpu as pltpu
from jax.experimental.pallas import tpu_sc as plsc

NC = 2
NS = 16
NW = NC * NS
CHUNK = 128

_MESH = plsc.VectorSubcoreMesh(
    core_axis_name="c", subcore_axis_name="s", num_cores=NC, num_subcores=NS
)


def _mm_body(x_ref, w_ref, o_ref):
    o_ref[...] = jnp.dot(x_ref[...], w_ref[...],
                         preferred_element_type=jnp.float32)


def _matmul(x, w):
    return pl.pallas_call(
        _mm_body,
        out_shape=jax.ShapeDtypeStruct((x.shape[0], w.shape[1]), jnp.float32),
    )(x, w)


def _mm_scale_body(x_ref, w_ref, d0_ref, d1_ref, o_ref):
    dinv = lax.rsqrt(d0_ref[...] + d1_ref[...] + 1.0)
    h = jnp.dot(x_ref[...], w_ref[...], preferred_element_type=jnp.float32)
    o_ref[...] = h * dinv


def _matmul_scale(x, w, d0, d1):
    return pl.pallas_call(
        _mm_scale_body,
        out_shape=jax.ShapeDtypeStruct((x.shape[0], w.shape[1]), jnp.float32),
    )(x, w, d0, d1)


def _layer2_body(p0_ref, p1_ref, g1_ref, d0_ref, d1_ref, b1_ref, w2_ref,
                 o_ref):
    dinv = lax.rsqrt(d0_ref[...] + d1_ref[...] + 1.0)
    h1 = (p0_ref[...] + p1_ref[...] + g1_ref[...]) * dinv + b1_ref[...]
    h1 = jnp.maximum(h1, 0.0)
    h1w = jnp.dot(h1, w2_ref[...], preferred_element_type=jnp.float32)
    o_ref[...] = h1w * dinv


def _layer2(p0, p1, g1, d0, d1, b1, w2):
    return pl.pallas_call(
        _layer2_body,
        out_shape=jax.ShapeDtypeStruct((g1.shape[0], w2.shape[1]),
                                       jnp.float32),
    )(p0, p1, g1, d0, d1, b1, w2)


def _head_body(q0_ref, q1_ref, g2_ref, d0_ref, d1_ref, b2_ref, wc1_ref,
               bc1_ref, wc2_ref, bc2_ref, o_ref):
    dinv = lax.rsqrt(d0_ref[...] + d1_ref[...] + 1.0)
    h2 = (q0_ref[...] + q1_ref[...] + g2_ref[...]) * dinv + b2_ref[...]
    z = jnp.dot(h2, wc1_ref[...], preferred_element_type=jnp.float32)
    z = jnp.maximum(z + bc1_ref[...], 0.0)
    o = jnp.dot(z, wc2_ref[...], preferred_element_type=jnp.float32)
    o = o + bc2_ref[...]
    o_ref[...] = 1.0 / (1.0 + jnp.exp(-o))


def _head(q0, q1, g2, d0, d1, b2, wc1, bc1, wc2, bc2):
    return pl.pallas_call(
        _head_body,
        out_shape=jax.ShapeDtypeStruct((g2.shape[0], wc2.shape[1]),
                                       jnp.float32),
    )(q0, q1, g2, d0, d1, b2, wc1, bc1, wc2, bc2)


def _make_deg_kernel(n_acc, chunks, slab=1):
    rows_per_tile = n_acc // NS
    assert chunks % slab == 0
    slabs = chunks // slab

    @functools.partial(
        pl.kernel,
        out_type=jax.ShapeDtypeStruct((NC, n_acc, 1), jnp.float32),
        mesh=_MESH,
        scratch_types=[
            pltpu.VMEM((slabs, slab * CHUNK), jnp.int32),
            pltpu.VMEM((slab * CHUNK, 1), jnp.float32),
            pltpu.VMEM_SHARED((n_acc, 1), jnp.float32),
        ],
        compiler_params=pltpu.CompilerParams(use_tc_tiling_on_sc=False),
    )
    def deg_kernel(didx_hbm, ones_hbm, zeros_hbm, out_hbm,
                   didx_v, ones_v, acc_sh):
        c = lax.axis_index("c")
        s = lax.axis_index("s")
        wid = c * NS + s
        tile_lo = s * rows_per_tile
        pltpu.sync_copy(zeros_hbm.at[pl.ds(tile_lo, rows_per_tile)],
                        acc_sh.at[pl.ds(tile_lo, rows_per_tile)])
        pltpu.sync_copy(didx_hbm.at[wid], didx_v)
        pltpu.sync_copy(ones_hbm, ones_v)
        plsc.subcore_barrier()

        def body(j, carry):
            pltpu.sync_copy(ones_v, acc_sh.at[didx_v.at[j]], add=True)
            return carry

        lax.fori_loop(0, slabs, body, 0)
        plsc.subcore_barrier()
        pltpu.sync_copy(acc_sh.at[pl.ds(tile_lo, rows_per_tile)],
                        out_hbm.at[c].at[pl.ds(tile_lo, rows_per_tile)])

    return deg_kernel


def _make_scatter_kernel(n_acc, chunks, d, slab=1):
    rows_per_tile = n_acc // NS
    assert chunks % slab == 0
    slabs = chunks // slab

    @functools.partial(
        pl.kernel,
        out_type=jax.ShapeDtypeStruct((NC, n_acc, d), jnp.float32),
        mesh=_MESH,
        scratch_types=[
            pltpu.VMEM((slabs, slab * CHUNK), jnp.int32),
            pltpu.VMEM((slabs, slab * CHUNK), jnp.int32),
            pltpu.VMEM((slab * CHUNK, d), jnp.float32),
            pltpu.VMEM_SHARED((n_acc, d), jnp.float32),
            pltpu.SemaphoreType.DMA,
        ],
        compiler_params=pltpu.CompilerParams(use_tc_tiling_on_sc=False),
    )
    def scatter_kernel(g_hbm, sidx_hbm, didx_hbm, zeros_hbm, out_hbm,
                       sidx_v, didx_v, rows_v, acc_sh, sem):
        c = lax.axis_index("c")
        s = lax.axis_index("s")
        wid = c * NS + s
        tile_lo = s * rows_per_tile
        pltpu.sync_copy(sidx_hbm.at[wid], sidx_v)
        pltpu.sync_copy(zeros_hbm.at[pl.ds(tile_lo, rows_per_tile)],
                        acc_sh.at[pl.ds(tile_lo, rows_per_tile)])
        pltpu.sync_copy(didx_hbm.at[wid], didx_v)
        plsc.subcore_barrier()

        def body(j, carry):
            pltpu.async_copy(g_hbm.at[sidx_v.at[j]], rows_v, sem).wait()
            pltpu.sync_copy(rows_v, acc_sh.at[didx_v.at[j]], add=True)
            return carry

        lax.fori_loop(0, slabs, body, 0)
        plsc.subcore_barrier()
        pltpu.sync_copy(acc_sh.at[pl.ds(tile_lo, rows_per_tile)],
                        out_hbm.at[c].at[pl.ds(tile_lo, rows_per_tile)])

    return scatter_kernel


def kernel(x, edge_index, W1, b1, W2, b2, Wc1, bc1, Wc2, bc2):
    n = x.shape[0]
    e = edge_index.shape[1]

    chunks = -(-e // (NW * CHUNK))
    chunks = -(-chunks // 4) * 4
    epw = chunks * CHUNK
    e_pad = epw * NW
    n_acc = -(-(n + 8) // (NS * 8)) * (NS * 8)

    src = edge_index[0].astype(jnp.int32)
    dst = edge_index[1].astype(jnp.int32)
    pad = e_pad - e
    src_p = jnp.concatenate([src, jnp.zeros((pad,), jnp.int32)])
    dst_p = jnp.concatenate([dst, jnp.full((pad,), n, jnp.int32)])
    slab = 1
    src_p = src_p.reshape(NW, chunks // slab, slab * CHUNK)
    dst_p = dst_p.reshape(NW, chunks // slab, slab * CHUNK)

    ones_col = jnp.ones((slab * CHUNK, 1), jnp.float32)
    zeros_col = jnp.zeros((n_acc, 1), jnp.float32)
    zeros_tab = jnp.zeros((n_acc, W1.shape[1]), jnp.float32)

    deg_kernel = _make_deg_kernel(n_acc, chunks)
    scat_kernel = _make_scatter_kernel(n_acc, chunks, W1.shape[1])

    deg_parts = deg_kernel(dst_p, ones_col, zeros_col)
    d0 = deg_parts[0, :n, :]
    d1 = deg_parts[1, :n, :]

    g1 = _matmul_scale(x, W1, d0, d1)
    parts1 = scat_kernel(g1, src_p, dst_p, zeros_tab)
    p0 = parts1[0, :n, :]
    p1 = parts1[1, :n, :]

    g2 = _layer2(p0, p1, g1, d0, d1, b1.reshape(1, -1), W2)
    parts2 = scat_kernel(g2, src_p, dst_p, zeros_tab)
    q0 = parts2[0, :n, :]
    q1 = parts2[1, :n, :]

    out = _head(q0, q1, g2, d0, d1, b2.reshape(1, -1), Wc1,
                bc1.reshape(1, -1), Wc2, bc2.reshape(1, -1))
    return out


# speedup vs baseline: 1.0343x; 1.0343x over previous
import functools

import jax
import jax.numpy as jnp
from jax import lax
from jax.experimental import pallas as pl
from jax.experimental.pallas import tpu as pltpu
from jax.experimental.pallas import tpu_sc as plsc

NC = 2
NS = 16
NW = NC * NS
CHUNK = 128

_MESH = plsc.VectorSubcoreMesh(
    core_axis_name="c", subcore_axis_name="s", num_cores=NC, num_subcores=NS
)


def _mm_body(x_ref, w_ref, o_ref):
    o_ref[...] = jnp.dot(x_ref[...], w_ref[...],
                         preferred_element_type=jnp.float32)


def _matmul(x, w):
    return pl.pallas_call(
        _mm_body,
        out_shape=jax.ShapeDtypeStruct((x.shape[0], w.shape[1]), jnp.float32),
    )(x, w)


def _norm_body(h_ref, d0_ref, d1_ref, o_ref):
    dinv = lax.rsqrt(d0_ref[...] + d1_ref[...] + 1.0)
    o_ref[...] = h_ref[...] * dinv


def _scale_by_dinv(h, d0, d1):
    return pl.pallas_call(
        _norm_body,
        out_shape=jax.ShapeDtypeStruct(h.shape, jnp.float32),
    )(h, d0, d1)


def _layer2_body(p0_ref, p1_ref, g1_ref, d0_ref, d1_ref, b1_ref, w2_ref,
                 o_ref):
    dinv = lax.rsqrt(d0_ref[...] + d1_ref[...] + 1.0)
    h1 = (p0_ref[...] + p1_ref[...] + g1_ref[...]) * dinv + b1_ref[...]
    h1 = jnp.maximum(h1, 0.0)
    h1w = jnp.dot(h1, w2_ref[...], preferred_element_type=jnp.float32)
    o_ref[...] = h1w * dinv


def _layer2(p0, p1, g1, d0, d1, b1, w2):
    return pl.pallas_call(
        _layer2_body,
        out_shape=jax.ShapeDtypeStruct((g1.shape[0], w2.shape[1]),
                                       jnp.float32),
    )(p0, p1, g1, d0, d1, b1, w2)


def _head_body(q0_ref, q1_ref, g2_ref, d0_ref, d1_ref, b2_ref, wc1_ref,
               bc1_ref, wc2_ref, bc2_ref, o_ref):
    dinv = lax.rsqrt(d0_ref[...] + d1_ref[...] + 1.0)
    h2 = (q0_ref[...] + q1_ref[...] + g2_ref[...]) * dinv + b2_ref[...]
    z = jnp.dot(h2, wc1_ref[...], preferred_element_type=jnp.float32)
    z = jnp.maximum(z + bc1_ref[...], 0.0)
    o = jnp.dot(z, wc2_ref[...], preferred_element_type=jnp.float32)
    o = o + bc2_ref[...]
    o_ref[...] = 1.0 / (1.0 + jnp.exp(-o))


def _head(q0, q1, g2, d0, d1, b2, wc1, bc1, wc2, bc2):
    return pl.pallas_call(
        _head_body,
        out_shape=jax.ShapeDtypeStruct((g2.shape[0], wc2.shape[1]),
                                       jnp.float32),
    )(q0, q1, g2, d0, d1, b2, wc1, bc1, wc2, bc2)


def _make_deg_kernel(n_acc, chunks, slab=1):
    rows_per_tile = n_acc // NS
    assert chunks % slab == 0
    slabs = chunks // slab

    @functools.partial(
        pl.kernel,
        out_type=jax.ShapeDtypeStruct((NC, n_acc, 1), jnp.float32),
        mesh=_MESH,
        scratch_types=[
            pltpu.VMEM((slabs, slab * CHUNK), jnp.int32),
            pltpu.VMEM((slab * CHUNK, 1), jnp.float32),
            pltpu.VMEM_SHARED((n_acc, 1), jnp.float32),
        ],
        compiler_params=pltpu.CompilerParams(use_tc_tiling_on_sc=False),
    )
    def deg_kernel(didx_hbm, ones_hbm, zeros_hbm, out_hbm,
                   didx_v, ones_v, acc_sh):
        c = lax.axis_index("c")
        s = lax.axis_index("s")
        wid = c * NS + s
        tile_lo = s * rows_per_tile
        pltpu.sync_copy(zeros_hbm.at[pl.ds(tile_lo, rows_per_tile)],
                        acc_sh.at[pl.ds(tile_lo, rows_per_tile)])
        pltpu.sync_copy(didx_hbm.at[wid], didx_v)
        pltpu.sync_copy(ones_hbm, ones_v)
        plsc.subcore_barrier()

        def body(j, carry):
            pltpu.sync_copy(ones_v, acc_sh.at[didx_v.at[j]], add=True)
            return carry

        lax.fori_loop(0, slabs, body, 0)
        plsc.subcore_barrier()
        pltpu.sync_copy(acc_sh.at[pl.ds(tile_lo, rows_per_tile)],
                        out_hbm.at[c].at[pl.ds(tile_lo, rows_per_tile)])

    return deg_kernel


def _make_scatter_kernel(n_acc, chunks, d, slab=1):
    rows_per_tile = n_acc // NS
    assert chunks % slab == 0
    slabs = chunks // slab

    @functools.partial(
        pl.kernel,
        out_type=jax.ShapeDtypeStruct((NC, n_acc, d), jnp.float32),
        mesh=_MESH,
        scratch_types=[
            pltpu.VMEM((slabs, slab * CHUNK), jnp.int32),
            pltpu.VMEM((slabs, slab * CHUNK), jnp.int32),
            pltpu.VMEM((slab * CHUNK, d), jnp.float32),
            pltpu.VMEM_SHARED((n_acc, d), jnp.float32),
            pltpu.SemaphoreType.DMA,
        ],
        compiler_params=pltpu.CompilerParams(use_tc_tiling_on_sc=False),
    )
    def scatter_kernel(g_hbm, sidx_hbm, didx_hbm, zeros_hbm, out_hbm,
                       sidx_v, didx_v, rows_v, acc_sh, sem):
        c = lax.axis_index("c")
        s = lax.axis_index("s")
        wid = c * NS + s
        tile_lo = s * rows_per_tile
        pltpu.sync_copy(sidx_hbm.at[wid], sidx_v)
        pltpu.sync_copy(zeros_hbm.at[pl.ds(tile_lo, rows_per_tile)],
                        acc_sh.at[pl.ds(tile_lo, rows_per_tile)])
        pltpu.sync_copy(didx_hbm.at[wid], didx_v)
        plsc.subcore_barrier()

        def body(j, carry):
            pltpu.async_copy(g_hbm.at[sidx_v.at[j]], rows_v, sem).wait()
            pltpu.sync_copy(rows_v, acc_sh.at[didx_v.at[j]], add=True)
            return carry

        lax.fori_loop(0, slabs, body, 0)
        plsc.subcore_barrier()
        pltpu.sync_copy(acc_sh.at[pl.ds(tile_lo, rows_per_tile)],
                        out_hbm.at[c].at[pl.ds(tile_lo, rows_per_tile)])

    return scatter_kernel


def kernel(x, edge_index, W1, b1, W2, b2, Wc1, bc1, Wc2, bc2):
    n = x.shape[0]
    e = edge_index.shape[1]

    chunks = -(-e // (NW * CHUNK))
    chunks = -(-chunks // 4) * 4
    epw = chunks * CHUNK
    e_pad = epw * NW
    n_acc = -(-(n + 8) // (NS * 8)) * (NS * 8)

    src = edge_index[0].astype(jnp.int32)
    dst = edge_index[1].astype(jnp.int32)
    pad = e_pad - e
    pad_dst = n + jnp.arange(pad, dtype=jnp.int32) % (n_acc - n)
    src_p = jnp.concatenate([src, jnp.zeros((pad,), jnp.int32)])
    dst_p = jnp.concatenate([dst, pad_dst])
    slab = 1
    src_p = src_p.reshape(NW, chunks // slab, slab * CHUNK)
    dst_p = dst_p.reshape(NW, chunks // slab, slab * CHUNK)

    ones_col = jnp.ones((slab * CHUNK, 1), jnp.float32)
    zeros_col = jnp.zeros((n_acc, 1), jnp.float32)
    zeros_tab = jnp.zeros((n_acc, W1.shape[1]), jnp.float32)

    deg_kernel = _make_deg_kernel(n_acc, chunks)
    scat_kernel = _make_scatter_kernel(n_acc, chunks, W1.shape[1])

    deg_parts = deg_kernel(dst_p, ones_col, zeros_col)
    d0 = deg_parts[0, :n, :]
    d1 = deg_parts[1, :n, :]

    h1w = _matmul(x, W1)
    g1 = _scale_by_dinv(h1w, d0, d1)
    parts1 = scat_kernel(g1, src_p, dst_p, zeros_tab)
    p0 = parts1[0, :n, :]
    p1 = parts1[1, :n, :]

    g2 = _layer2(p0, p1, g1, d0, d1, b1.reshape(1, -1), W2)
    parts2 = scat_kernel(g2, src_p, dst_p, zeros_tab)
    q0 = parts2[0, :n, :]
    q1 = parts2[1, :n, :]

    out = _head(q0, q1, g2, d0, d1, b2.reshape(1, -1), Wc1,
                bc1.reshape(1, -1), Wc2, bc2.reshape(1, -1))
    return out


# speedup vs baseline: 1.4023x; 1.3558x over previous
import functools

import jax
import jax.numpy as jnp
from jax import lax
from jax.experimental import pallas as pl
from jax.experimental.pallas import tpu as pltpu
from jax.experimental.pallas import tpu_sc as plsc

NC = 2
NS = 16
NW = NC * NS
CHUNK = 128

_MESH = plsc.VectorSubcoreMesh(
    core_axis_name="c", subcore_axis_name="s", num_cores=NC, num_subcores=NS
)


def _mm_body(x_ref, w_ref, o_ref):
    o_ref[...] = jnp.dot(x_ref[...], w_ref[...],
                         preferred_element_type=jnp.float32)


def _matmul(x, w):
    return pl.pallas_call(
        _mm_body,
        out_shape=jax.ShapeDtypeStruct((x.shape[0], w.shape[1]), jnp.float32),
    )(x, w)


def _norm_body(h_ref, d0_ref, d1_ref, o_ref):
    dinv = lax.rsqrt(d0_ref[...] + d1_ref[...] + 1.0)
    o_ref[...] = h_ref[...] * dinv


def _scale_by_dinv(h, d0, d1):
    return pl.pallas_call(
        _norm_body,
        out_shape=jax.ShapeDtypeStruct(h.shape, jnp.float32),
    )(h, d0, d1)


def _layer2_body(p0_ref, p1_ref, g1_ref, d0_ref, d1_ref, b1_ref, w2_ref,
                 o_ref):
    dinv = lax.rsqrt(d0_ref[...] + d1_ref[...] + 1.0)
    h1 = (p0_ref[...] + p1_ref[...] + g1_ref[...]) * dinv + b1_ref[...]
    h1 = jnp.maximum(h1, 0.0)
    h1w = jnp.dot(h1, w2_ref[...], preferred_element_type=jnp.float32)
    o_ref[...] = h1w * dinv


def _layer2(p0, p1, g1, d0, d1, b1, w2):
    return pl.pallas_call(
        _layer2_body,
        out_shape=jax.ShapeDtypeStruct((g1.shape[0], w2.shape[1]),
                                       jnp.float32),
    )(p0, p1, g1, d0, d1, b1, w2)


def _head_body(q0_ref, q1_ref, g2_ref, d0_ref, d1_ref, b2_ref, wc1_ref,
               bc1_ref, wc2_ref, bc2_ref, o_ref):
    dinv = lax.rsqrt(d0_ref[...] + d1_ref[...] + 1.0)
    h2 = (q0_ref[...] + q1_ref[...] + g2_ref[...]) * dinv + b2_ref[...]
    z = jnp.dot(h2, wc1_ref[...], preferred_element_type=jnp.float32)
    z = jnp.maximum(z + bc1_ref[...], 0.0)
    o = jnp.dot(z, wc2_ref[...], preferred_element_type=jnp.float32)
    o = o + bc2_ref[...]
    o_ref[...] = 1.0 / (1.0 + jnp.exp(-o))


def _head(q0, q1, g2, d0, d1, b2, wc1, bc1, wc2, bc2):
    return pl.pallas_call(
        _head_body,
        out_shape=jax.ShapeDtypeStruct((g2.shape[0], wc2.shape[1]),
                                       jnp.float32),
    )(q0, q1, g2, d0, d1, b2, wc1, bc1, wc2, bc2)


def _make_deg_kernel(n_acc, chunks, slab=1):
    rows_per_tile = n_acc // NS
    assert chunks % slab == 0
    slabs = chunks // slab

    @functools.partial(
        pl.kernel,
        out_type=jax.ShapeDtypeStruct((NC, n_acc, 1), jnp.float32),
        mesh=_MESH,
        scratch_types=[
            pltpu.VMEM((slabs, slab * CHUNK), jnp.int32),
            pltpu.VMEM((slab * CHUNK, 1), jnp.float32),
            pltpu.VMEM_SHARED((n_acc, 1), jnp.float32),
        ],
        compiler_params=pltpu.CompilerParams(use_tc_tiling_on_sc=False),
    )
    def deg_kernel(didx_hbm, ones_hbm, zeros_hbm, out_hbm,
                   didx_v, ones_v, acc_sh):
        c = lax.axis_index("c")
        s = lax.axis_index("s")
        wid = c * NS + s
        tile_lo = s * rows_per_tile
        pltpu.sync_copy(zeros_hbm.at[pl.ds(tile_lo, rows_per_tile)],
                        acc_sh.at[pl.ds(tile_lo, rows_per_tile)])
        pltpu.sync_copy(didx_hbm.at[wid], didx_v)
        pltpu.sync_copy(ones_hbm, ones_v)
        plsc.subcore_barrier()

        def body(j, carry):
            pltpu.sync_copy(ones_v, acc_sh.at[didx_v.at[j]], add=True)
            return carry

        lax.fori_loop(0, slabs, body, 0)
        plsc.subcore_barrier()
        pltpu.sync_copy(acc_sh.at[pl.ds(tile_lo, rows_per_tile)],
                        out_hbm.at[c].at[pl.ds(tile_lo, rows_per_tile)])

    return deg_kernel


def _make_scatter_kernel(n_acc, chunks, d, slab=1):
    rows_per_tile = n_acc // NS
    assert chunks % slab == 0
    slabs = chunks // slab

    @functools.partial(
        pl.kernel,
        out_type=jax.ShapeDtypeStruct((NC, n_acc, d), jnp.float32),
        mesh=_MESH,
        scratch_types=[
            pltpu.VMEM((slabs, slab * CHUNK), jnp.int32),
            pltpu.VMEM((slabs, slab * CHUNK), jnp.int32),
            pltpu.VMEM((slab * CHUNK, d), jnp.float32),
            pltpu.VMEM_SHARED((n_acc, d), jnp.float32),
            pltpu.SemaphoreType.DMA,
        ],
        compiler_params=pltpu.CompilerParams(use_tc_tiling_on_sc=False),
    )
    def scatter_kernel(g_hbm, sidx_hbm, didx_hbm, zeros_hbm, out_hbm,
                       sidx_v, didx_v, rows_v, acc_sh, sem):
        c = lax.axis_index("c")
        s = lax.axis_index("s")
        wid = c * NS + s
        tile_lo = s * rows_per_tile
        pltpu.sync_copy(sidx_hbm.at[wid], sidx_v)
        pltpu.sync_copy(zeros_hbm.at[pl.ds(tile_lo, rows_per_tile)],
                        acc_sh.at[pl.ds(tile_lo, rows_per_tile)])
        pltpu.sync_copy(didx_hbm.at[wid], didx_v)
        plsc.subcore_barrier()

        def body(j, carry):
            pltpu.async_copy(g_hbm.at[sidx_v.at[j]], rows_v, sem).wait()
            pltpu.sync_copy(rows_v, acc_sh.at[didx_v.at[j]], add=True)
            return carry

        lax.fori_loop(0, slabs, body, 0)
        plsc.subcore_barrier()
        pltpu.sync_copy(acc_sh.at[pl.ds(tile_lo, rows_per_tile)],
                        out_hbm.at[c].at[pl.ds(tile_lo, rows_per_tile)])

    return scatter_kernel


def kernel(x, edge_index, W1, b1, W2, b2, Wc1, bc1, Wc2, bc2):
    n = x.shape[0]
    e = edge_index.shape[1]

    chunks = -(-e // (NW * CHUNK))
    epw = chunks * CHUNK
    e_pad = epw * NW
    n_acc = -(-(n + 8) // (NS * 8)) * (NS * 8)

    src = edge_index[0].astype(jnp.int32)
    dst = edge_index[1].astype(jnp.int32)
    pad = e_pad - e
    pad_dst = n + jnp.arange(pad, dtype=jnp.int32) % (n_acc - n)
    src_p = jnp.concatenate([src, jnp.zeros((pad,), jnp.int32)])
    dst_p = jnp.concatenate([dst, pad_dst])
    slab = 1
    src_p = src_p.reshape(NW, chunks // slab, slab * CHUNK)
    dst_p = dst_p.reshape(NW, chunks // slab, slab * CHUNK)

    ones_col = jnp.ones((slab * CHUNK, 1), jnp.float32)
    zeros_col = jnp.zeros((n_acc, 1), jnp.float32)
    zeros_tab = jnp.zeros((n_acc, W1.shape[1]), jnp.float32)

    deg_kernel = _make_deg_kernel(n_acc, chunks)
    scat_kernel = _make_scatter_kernel(n_acc, chunks, W1.shape[1])

    deg_parts = deg_kernel(dst_p, ones_col, zeros_col)
    d0 = deg_parts[0, :n, :]
    d1 = deg_parts[1, :n, :]

    h1w = _matmul(x, W1)
    g1 = _scale_by_dinv(h1w, d0, d1)
    parts1 = scat_kernel(g1, src_p, dst_p, zeros_tab)
    p0 = parts1[0, :n, :]
    p1 = parts1[1, :n, :]

    g2 = _layer2(p0, p1, g1, d0, d1, b1.reshape(1, -1), W2)
    parts2 = scat_kernel(g2, src_p, dst_p, zeros_tab)
    q0 = parts2[0, :n, :]
    q1 = parts2[1, :n, :]

    out = _head(q0, q1, g2, d0, d1, b2.reshape(1, -1), Wc1,
                bc1.reshape(1, -1), Wc2, bc2.reshape(1, -1))
    return out
